# Initial kernel scaffold; baseline (speedup 1.0000x reference)
import jax
import jax.numpy as jnp
from jax.experimental import pallas as pl


def kernel(x, edge_index, W1, b1, W2, b2):
    raise NotImplementedError("write your pallas kernel here")


# speedup vs baseline: 55.9545x; 55.9545x over previous
import functools

import jax
import jax.numpy as jnp
from jax import lax
from jax.experimental import pallas as pl
from jax.experimental.pallas import tpu as pltpu
from jax.experimental.pallas import tpu_sc as plsc

N = 10000
E = 320000
D = 128
H = 32
C = 2

NC = 2
NS = 16
NW = NC * NS
CHUNK = 128
E_PAD = 327680
CPT = E_PAD // (NW * CHUNK)
EPT = CPT * CHUNK
NP = 10240
RPT = NP // NS

_mesh = plsc.VectorSubcoreMesh(core_axis_name="c", subcore_axis_name="s")


@functools.partial(
    pl.kernel,
    mesh=_mesh,
    out_type=jax.ShapeDtypeStruct((NW, NP), jnp.float32),
    scratch_types=[
        pltpu.VMEM((EPT,), jnp.int32),
        pltpu.VMEM((NP,), jnp.float32),
    ],
    compiler_params=pltpu.CompilerParams(needs_layout_passes=False),
)
def _deg_kernel(dst_hbm, out_hbm, dst_v, deg_v):
    c = lax.axis_index("c")
    s = lax.axis_index("s")
    wid = s * NC + c

    pltpu.sync_copy(dst_hbm.at[pl.ds(wid * EPT, EPT)], dst_v)

    zeros = jnp.zeros((16,), jnp.float32)

    def _zero(i, _):
        deg_v[pl.ds(i * 16, 16)] = zeros
        return 0

    lax.fori_loop(0, NP // 16, _zero, 0)

    ones = jnp.full((16,), 1.0, jnp.float32)

    def _count(i, _):
        idx = dst_v[pl.ds(i * 16, 16)]
        plsc.addupdate_scatter(deg_v, [idx], ones)
        return 0

    lax.fori_loop(0, EPT // 16, _count, 0)

    pltpu.sync_copy(deg_v, out_hbm.at[wid])


@functools.partial(
    pl.kernel,
    mesh=_mesh,
    out_type=jax.ShapeDtypeStruct((NC, NP, H), jnp.float32),
    scratch_types=[
        pltpu.VMEM((CPT, CHUNK), jnp.int32),
        pltpu.VMEM((CPT, CHUNK), jnp.int32),
        [pltpu.VMEM((CHUNK, H), jnp.float32) for _ in range(8)],
        pltpu.VMEM_SHARED((NP, H), jnp.float32),
        pltpu.VMEM_SHARED((NP, H), jnp.float32),
        [pltpu.SemaphoreType.DMA for _ in range(8)],
        [pltpu.SemaphoreType.DMA for _ in range(8)],
    ],
    compiler_params=pltpu.CompilerParams(use_tc_tiling_on_sc=False),
)
def _seg_kernel(table_hbm, src_hbm, dst_hbm, zrows_hbm, out_hbm,
                src_v, dst_v, rows, acc_sh, tab_sh, gsem, ssem):
    c = lax.axis_index("c")
    s = lax.axis_index("s")
    wid = s * NC + c
    NB = 4

    pltpu.sync_copy(src_hbm.at[pl.ds(wid * CPT, CPT)], src_v)
    pltpu.sync_copy(dst_hbm.at[pl.ds(wid * CPT, CPT)], dst_v)
    pltpu.sync_copy(table_hbm.at[pl.ds(s * RPT, RPT)],
                    tab_sh.at[pl.ds(s * RPT, RPT)])
    pltpu.sync_copy(zrows_hbm, acc_sh.at[pl.ds(s * RPT, RPT)])
    plsc.subcore_barrier()

    NR = CPT // (2 * NB)
    bank_a = rows[:NB]
    bank_b = rows[NB:]

    def _gather(buf, gs, j):
        pltpu.async_copy(tab_sh.at[src_v.at[j]], buf, gs)

    def _wait_gather(buf, gs, j):
        pltpu.make_async_copy(tab_sh.at[src_v.at[j]], buf, gs).wait()

    def _scatter(buf, ss, j):
        pltpu.async_copy(buf, acc_sh.at[dst_v.at[j]], ss, add=True)

    def _wait_scatter(buf, ss, j):
        pltpu.make_async_copy(buf, acc_sh.at[dst_v.at[j]], ss).wait()

    for b in range(NB):
        _gather(bank_a[b], gsem[b], b)

    def _round(P, _):
        j0 = 2 * NB * P
        for b in range(NB):
            _wait_gather(bank_a[b], gsem[b], j0 + b)
            _scatter(bank_a[b], ssem[b], j0 + b)
        for b in range(NB):
            @pl.when(P > 0)
            def _():
                _wait_scatter(bank_b[b], ssem[NB + b], j0 - NB + b)
            _gather(bank_b[b], gsem[NB + b], j0 + NB + b)
        for b in range(NB):
            _wait_gather(bank_b[b], gsem[NB + b], j0 + NB + b)
            _scatter(bank_b[b], ssem[NB + b], j0 + NB + b)
        for b in range(NB):
            @pl.when(P < NR - 1)
            def _():
                _wait_scatter(bank_a[b], ssem[b], j0 + b)
                _gather(bank_a[b], gsem[b], j0 + 2 * NB + b)
        return 0

    lax.fori_loop(0, NR, _round, 0)

    for b in range(NB):
        _wait_scatter(bank_a[b], ssem[b], CPT - 2 * NB + b)
        _wait_scatter(bank_b[b], ssem[NB + b], CPT - NB + b)

    plsc.subcore_barrier()
    pltpu.sync_copy(acc_sh.at[pl.ds(s * RPT, RPT)],
                    out_hbm.at[c, pl.ds(s * RPT, RPT)])


@functools.partial(
    pl.kernel,
    mesh=_mesh,
    out_type=jax.ShapeDtypeStruct((NW, C, NP), jnp.float32),
    scratch_types=[
        pltpu.VMEM((EPT,), jnp.int32),
        pltpu.VMEM((EPT,), jnp.int32),
        pltpu.VMEM((NP,), jnp.float32),
        pltpu.VMEM((NP,), jnp.float32),
        pltpu.VMEM((NP,), jnp.float32),
        pltpu.VMEM((NP,), jnp.float32),
    ],
    compiler_params=pltpu.CompilerParams(needs_layout_passes=False),
)
def _l2_kernel(t2t_hbm, src_hbm, dst_hbm, out_hbm,
               src_v, dst_v, tab0, tab1, acc0, acc1):
    c = lax.axis_index("c")
    s = lax.axis_index("s")
    wid = s * NC + c

    pltpu.sync_copy(src_hbm.at[pl.ds(wid * EPT, EPT)], src_v)
    pltpu.sync_copy(dst_hbm.at[pl.ds(wid * EPT, EPT)], dst_v)
    pltpu.sync_copy(t2t_hbm.at[0], tab0)
    pltpu.sync_copy(t2t_hbm.at[1], tab1)

    zeros = jnp.zeros((16,), jnp.float32)

    def _zero(i, _):
        acc0[pl.ds(i * 16, 16)] = zeros
        acc1[pl.ds(i * 16, 16)] = zeros
        return 0

    lax.fori_loop(0, NP // 16, _zero, 0)

    def _edges(e, _):
        sv = src_v[pl.ds(e * 16, 16)]
        dv = dst_v[pl.ds(e * 16, 16)]
        v0 = plsc.load_gather(tab0, [sv])
        plsc.addupdate_scatter(acc0, [dv], v0)
        v1 = plsc.load_gather(tab1, [sv])
        plsc.addupdate_scatter(acc1, [dv], v1)
        return 0

    lax.fori_loop(0, EPT // 16, _edges, 0)

    pltpu.sync_copy(acc0, out_hbm.at[wid, 0])
    pltpu.sync_copy(acc1, out_hbm.at[wid, 1])


BN = 1024


def _tc_b_body(degp_ref, x_ref, w1_ref, h1s_ref, dinv_ref):
    deg = jnp.sum(degp_ref[...], axis=0) + 1.0
    dinv = lax.rsqrt(deg)
    h = jnp.dot(x_ref[...], w1_ref[...], preferred_element_type=jnp.float32)
    h1s_ref[...] = h * dinv[:, None]
    dinv_ref[...] = dinv[:, None]


def _tc_d_body(acc_ref, h1s_ref, dinv_ref, b1_ref, w2_ref, t2t_ref, *,
               nblock):
    i = pl.program_id(0)
    ssum = acc_ref[0] + acc_ref[1] + h1s_ref[...]
    dinv = dinv_ref[...]
    a = jnp.maximum(dinv * ssum + b1_ref[...], 0.0) * dinv
    row = i * nblock + lax.broadcasted_iota(jnp.int32, (nblock, 1), 0)
    a1s = jnp.where(row < N, a, 0.0)
    t2t_ref[...] = jnp.dot(a1s, w2_ref[...],
                           preferred_element_type=jnp.float32).T


def _tc_e_body(acc_ref, t2t_ref, dinvt_ref, b2_ref, out_ref):
    ssum = jnp.sum(acc_ref[...], axis=0)
    o = dinvt_ref[...] * (ssum + t2t_ref[...]) + b2_ref[...]
    m = jnp.max(o, axis=0, keepdims=True)
    lse = m + jnp.log(jnp.sum(jnp.exp(o - m), axis=0, keepdims=True))
    out_ref[...] = o - lse


def _tc_b(deg_part, x_p, w1):
    grid = NP // BN
    return pl.pallas_call(
        _tc_b_body,
        grid=(grid,),
        in_specs=[
            pl.BlockSpec((NW, BN), lambda i: (0, i)),
            pl.BlockSpec((BN, D), lambda i: (i, 0)),
            pl.BlockSpec((D, H), lambda i: (0, 0)),
        ],
        out_specs=[
            pl.BlockSpec((BN, H), lambda i: (i, 0)),
            pl.BlockSpec((BN, 1), lambda i: (i, 0)),
        ],
        out_shape=[
            jax.ShapeDtypeStruct((NP, H), jnp.float32),
            jax.ShapeDtypeStruct((NP, 1), jnp.float32),
        ],
    )(deg_part, x_p, w1)


def _tc_d(acc1, h1s, dinv, b1, w2):
    grid = NP // BN
    return pl.pallas_call(
        functools.partial(_tc_d_body, nblock=BN),
        grid=(grid,),
        in_specs=[
            pl.BlockSpec((NC, BN, H), lambda i: (0, i, 0)),
            pl.BlockSpec((BN, H), lambda i: (i, 0)),
            pl.BlockSpec((BN, 1), lambda i: (i, 0)),
            pl.BlockSpec((1, H), lambda i: (0, 0)),
            pl.BlockSpec((H, C), lambda i: (0, 0)),
        ],
        out_specs=pl.BlockSpec((C, BN), lambda i: (0, i)),
        out_shape=jax.ShapeDtypeStruct((C, NP), jnp.float32),
    )(acc1, h1s, dinv, b1, w2)


def _tc_e(acc2p, t2t, dinvt, b2):
    grid = NP // BN
    return pl.pallas_call(
        _tc_e_body,
        grid=(grid,),
        in_specs=[
            pl.BlockSpec((NW, C, BN), lambda i: (0, 0, i)),
            pl.BlockSpec((C, BN), lambda i: (0, i)),
            pl.BlockSpec((1, BN), lambda i: (0, i)),
            pl.BlockSpec((C, 1), lambda i: (0, 0)),
        ],
        out_specs=pl.BlockSpec((C, BN), lambda i: (0, i)),
        out_shape=jax.ShapeDtypeStruct((C, NP), jnp.float32),
    )(acc2p, t2t, dinvt, b2)


@jax.jit
def kernel(x, edge_index, W1, b1, W2, b2):
    src = edge_index[0]
    dst = edge_index[1]
    pad = jnp.full((E_PAD - E,), N, jnp.int32)
    src_p = jnp.concatenate([src, pad]).reshape(E_PAD // CHUNK, CHUNK)
    dst_p = jnp.concatenate([dst, pad]).reshape(E_PAD // CHUNK, CHUNK)
    src_flat = src_p.reshape(E_PAD)
    dst_flat = dst_p.reshape(E_PAD)
    x_p = jnp.pad(x, ((0, NP - N), (0, 0)))
    zrows = jnp.zeros((RPT, H), jnp.float32)

    deg_part = _deg_kernel(dst_flat)
    h1s, dinv = _tc_b(deg_part, x_p, W1)
    acc1 = _seg_kernel(h1s, src_p, dst_p, zrows)
    t2t = _tc_d(acc1, h1s, dinv, b1.reshape(1, H), W2)
    acc2p = _l2_kernel(t2t, src_flat, dst_flat)
    out_t = _tc_e(acc2p, t2t, dinv.reshape(1, NP), b2.reshape(C, 1))
    return out_t[:, :N].T


# speedup vs baseline: 57.0888x; 1.0203x over previous
import functools

import jax
import jax.numpy as jnp
from jax import lax
from jax.experimental import pallas as pl
from jax.experimental.pallas import tpu as pltpu
from jax.experimental.pallas import tpu_sc as plsc

N = 10000
E = 320000
D = 128
H = 32
C = 2

NC = 2
NS = 16
NW = NC * NS
CHUNK = 128
E_PAD = 327680
CPT = E_PAD // (NW * CHUNK)
EPT = CPT * CHUNK
NP = 10240
RPT = NP // NS

_mesh = plsc.VectorSubcoreMesh(core_axis_name="c", subcore_axis_name="s")


@functools.partial(
    pl.kernel,
    mesh=_mesh,
    out_type=jax.ShapeDtypeStruct((NW, NP), jnp.float32),
    scratch_types=[
        pltpu.VMEM((EPT,), jnp.int32),
        pltpu.VMEM((NP,), jnp.float32),
    ],
    compiler_params=pltpu.CompilerParams(needs_layout_passes=False),
)
def _deg_kernel(dst_hbm, out_hbm, dst_v, deg_v):
    c = lax.axis_index("c")
    s = lax.axis_index("s")
    wid = s * NC + c

    pltpu.sync_copy(dst_hbm.at[pl.ds(wid * EPT, EPT)], dst_v)

    zeros = jnp.zeros((16,), jnp.float32)

    def _zero(i, _):
        for u in range(4):
            deg_v[pl.ds(i * 64 + u * 16, 16)] = zeros
        return 0

    lax.fori_loop(0, NP // 64, _zero, 0)

    ones = jnp.full((16,), 1.0, jnp.float32)

    def _count(i, _):
        for u in range(4):
            idx = dst_v[pl.ds(i * 64 + u * 16, 16)]
            plsc.addupdate_scatter(deg_v, [idx], ones)
        return 0

    lax.fori_loop(0, EPT // 64, _count, 0)

    pltpu.sync_copy(deg_v, out_hbm.at[wid])


@functools.partial(
    pl.kernel,
    mesh=_mesh,
    out_type=jax.ShapeDtypeStruct((NC, NP, H), jnp.float32),
    scratch_types=[
        pltpu.VMEM((CPT, CHUNK), jnp.int32),
        pltpu.VMEM((CPT, CHUNK), jnp.int32),
        [pltpu.VMEM((CHUNK, H), jnp.float32) for _ in range(8)],
        pltpu.VMEM_SHARED((NP, H), jnp.float32),
        pltpu.VMEM_SHARED((NP, H), jnp.float32),
        [pltpu.SemaphoreType.DMA for _ in range(8)],
        [pltpu.SemaphoreType.DMA for _ in range(8)],
    ],
    compiler_params=pltpu.CompilerParams(use_tc_tiling_on_sc=False),
)
def _seg_kernel(table_hbm, src_hbm, dst_hbm, zrows_hbm, out_hbm,
                src_v, dst_v, rows, acc_sh, tab_sh, gsem, ssem):
    c = lax.axis_index("c")
    s = lax.axis_index("s")
    wid = s * NC + c
    NB = 4

    pltpu.sync_copy(src_hbm.at[pl.ds(wid * CPT, CPT)], src_v)
    pltpu.sync_copy(dst_hbm.at[pl.ds(wid * CPT, CPT)], dst_v)
    pltpu.sync_copy(table_hbm.at[pl.ds(s * RPT, RPT)],
                    tab_sh.at[pl.ds(s * RPT, RPT)])
    pltpu.sync_copy(zrows_hbm, acc_sh.at[pl.ds(s * RPT, RPT)])
    plsc.subcore_barrier()

    NR = CPT // (2 * NB)
    bank_a = rows[:NB]
    bank_b = rows[NB:]

    def _gather(buf, gs, j):
        pltpu.async_copy(tab_sh.at[src_v.at[j]], buf, gs)

    def _wait_gather(buf, gs, j):
        pltpu.make_async_copy(tab_sh.at[src_v.at[j]], buf, gs).wait()

    def _scatter(buf, ss, j):
        pltpu.async_copy(buf, acc_sh.at[dst_v.at[j]], ss, add=True)

    def _wait_scatter(buf, ss, j):
        pltpu.make_async_copy(buf, acc_sh.at[dst_v.at[j]], ss).wait()

    for b in range(NB):
        _gather(bank_a[b], gsem[b], b)

    def _round(P, _):
        j0 = 2 * NB * P
        for b in range(NB):
            _wait_gather(bank_a[b], gsem[b], j0 + b)
            _scatter(bank_a[b], ssem[b], j0 + b)
        for b in range(NB):
            @pl.when(P > 0)
            def _():
                _wait_scatter(bank_b[b], ssem[NB + b], j0 - NB + b)
            _gather(bank_b[b], gsem[NB + b], j0 + NB + b)
        for b in range(NB):
            _wait_gather(bank_b[b], gsem[NB + b], j0 + NB + b)
            _scatter(bank_b[b], ssem[NB + b], j0 + NB + b)
        for b in range(NB):
            @pl.when(P < NR - 1)
            def _():
                _wait_scatter(bank_a[b], ssem[b], j0 + b)
                _gather(bank_a[b], gsem[b], j0 + 2 * NB + b)
        return 0

    lax.fori_loop(0, NR, _round, 0)

    for b in range(NB):
        _wait_scatter(bank_a[b], ssem[b], CPT - 2 * NB + b)
        _wait_scatter(bank_b[b], ssem[NB + b], CPT - NB + b)

    plsc.subcore_barrier()
    pltpu.sync_copy(acc_sh.at[pl.ds(s * RPT, RPT)],
                    out_hbm.at[c, pl.ds(s * RPT, RPT)])


@functools.partial(
    pl.kernel,
    mesh=_mesh,
    out_type=jax.ShapeDtypeStruct((NW, C, NP), jnp.float32),
    scratch_types=[
        pltpu.VMEM((EPT,), jnp.int32),
        pltpu.VMEM((EPT,), jnp.int32),
        pltpu.VMEM((NP,), jnp.float32),
        pltpu.VMEM((NP,), jnp.float32),
        pltpu.VMEM((NP,), jnp.float32),
        pltpu.VMEM((NP,), jnp.float32),
    ],
    compiler_params=pltpu.CompilerParams(needs_layout_passes=False),
)
def _l2_kernel(t2t_hbm, src_hbm, dst_hbm, out_hbm,
               src_v, dst_v, tab0, tab1, acc0, acc1):
    c = lax.axis_index("c")
    s = lax.axis_index("s")
    wid = s * NC + c

    pltpu.sync_copy(src_hbm.at[pl.ds(wid * EPT, EPT)], src_v)
    pltpu.sync_copy(dst_hbm.at[pl.ds(wid * EPT, EPT)], dst_v)
    pltpu.sync_copy(t2t_hbm.at[0], tab0)
    pltpu.sync_copy(t2t_hbm.at[1], tab1)

    zeros = jnp.zeros((16,), jnp.float32)

    def _zero(i, _):
        for u in range(4):
            acc0[pl.ds(i * 64 + u * 16, 16)] = zeros
            acc1[pl.ds(i * 64 + u * 16, 16)] = zeros
        return 0

    lax.fori_loop(0, NP // 64, _zero, 0)

    def _edges(e, _):
        for u in range(4):
            sv = src_v[pl.ds(e * 64 + u * 16, 16)]
            dv = dst_v[pl.ds(e * 64 + u * 16, 16)]
            v0 = plsc.load_gather(tab0, [sv])
            plsc.addupdate_scatter(acc0, [dv], v0)
            v1 = plsc.load_gather(tab1, [sv])
            plsc.addupdate_scatter(acc1, [dv], v1)
        return 0

    lax.fori_loop(0, EPT // 64, _edges, 0)

    pltpu.sync_copy(acc0, out_hbm.at[wid, 0])
    pltpu.sync_copy(acc1, out_hbm.at[wid, 1])


BN = 1024


def _tc_b_body(degp_ref, x_ref, w1_ref, h1s_ref, dinv_ref):
    deg = jnp.sum(degp_ref[...], axis=0) + 1.0
    dinv = lax.rsqrt(deg)
    h = jnp.dot(x_ref[...], w1_ref[...], preferred_element_type=jnp.float32)
    h1s_ref[...] = h * dinv[:, None]
    dinv_ref[...] = dinv[:, None]


def _tc_d_body(acc_ref, h1s_ref, dinv_ref, b1_ref, w2_ref, t2t_ref, *,
               nblock):
    i = pl.program_id(0)
    ssum = acc_ref[0] + acc_ref[1] + h1s_ref[...]
    dinv = dinv_ref[...]
    a = jnp.maximum(dinv * ssum + b1_ref[...], 0.0) * dinv
    row = i * nblock + lax.broadcasted_iota(jnp.int32, (nblock, 1), 0)
    a1s = jnp.where(row < N, a, 0.0)
    t2t_ref[...] = jnp.dot(a1s, w2_ref[...],
                           preferred_element_type=jnp.float32).T


def _tc_e_body(acc_ref, t2t_ref, dinvt_ref, b2_ref, out_ref):
    ssum = jnp.sum(acc_ref[...], axis=0)
    o = dinvt_ref[...] * (ssum + t2t_ref[...]) + b2_ref[...]
    m = jnp.max(o, axis=0, keepdims=True)
    lse = m + jnp.log(jnp.sum(jnp.exp(o - m), axis=0, keepdims=True))
    out_ref[...] = o - lse


def _tc_b(deg_part, x_p, w1):
    grid = NP // BN
    return pl.pallas_call(
        _tc_b_body,
        grid=(grid,),
        in_specs=[
            pl.BlockSpec((NW, BN), lambda i: (0, i)),
            pl.BlockSpec((BN, D), lambda i: (i, 0)),
            pl.BlockSpec((D, H), lambda i: (0, 0)),
        ],
        out_specs=[
            pl.BlockSpec((BN, H), lambda i: (i, 0)),
            pl.BlockSpec((BN, 1), lambda i: (i, 0)),
        ],
        out_shape=[
            jax.ShapeDtypeStruct((NP, H), jnp.float32),
            jax.ShapeDtypeStruct((NP, 1), jnp.float32),
        ],
    )(deg_part, x_p, w1)


def _tc_d(acc1, h1s, dinv, b1, w2):
    grid = NP // BN
    return pl.pallas_call(
        functools.partial(_tc_d_body, nblock=BN),
        grid=(grid,),
        in_specs=[
            pl.BlockSpec((NC, BN, H), lambda i: (0, i, 0)),
            pl.BlockSpec((BN, H), lambda i: (i, 0)),
            pl.BlockSpec((BN, 1), lambda i: (i, 0)),
            pl.BlockSpec((1, H), lambda i: (0, 0)),
            pl.BlockSpec((H, C), lambda i: (0, 0)),
        ],
        out_specs=pl.BlockSpec((C, BN), lambda i: (0, i)),
        out_shape=jax.ShapeDtypeStruct((C, NP), jnp.float32),
    )(acc1, h1s, dinv, b1, w2)


def _tc_e(acc2p, t2t, dinvt, b2):
    grid = NP // BN
    return pl.pallas_call(
        _tc_e_body,
        grid=(grid,),
        in_specs=[
            pl.BlockSpec((NW, C, BN), lambda i: (0, 0, i)),
            pl.BlockSpec((C, BN), lambda i: (0, i)),
            pl.BlockSpec((1, BN), lambda i: (0, i)),
            pl.BlockSpec((C, 1), lambda i: (0, 0)),
        ],
        out_specs=pl.BlockSpec((C, BN), lambda i: (0, i)),
        out_shape=jax.ShapeDtypeStruct((C, NP), jnp.float32),
    )(acc2p, t2t, dinvt, b2)


@jax.jit
def kernel(x, edge_index, W1, b1, W2, b2):
    src = edge_index[0]
    dst = edge_index[1]
    pad = jnp.full((E_PAD - E,), N, jnp.int32)
    src_p = jnp.concatenate([src, pad]).reshape(E_PAD // CHUNK, CHUNK)
    dst_p = jnp.concatenate([dst, pad]).reshape(E_PAD // CHUNK, CHUNK)
    src_flat = src_p.reshape(E_PAD)
    dst_flat = dst_p.reshape(E_PAD)
    x_p = jnp.pad(x, ((0, NP - N), (0, 0)))
    zrows = jnp.zeros((RPT, H), jnp.float32)

    deg_part = _deg_kernel(dst_flat)
    h1s, dinv = _tc_b(deg_part, x_p, W1)
    acc1 = _seg_kernel(h1s, src_p, dst_p, zrows)
    t2t = _tc_d(acc1, h1s, dinv, b1.reshape(1, H), W2)
    acc2p = _l2_kernel(t2t, src_flat, dst_flat)
    out_t = _tc_e(acc2p, t2t, dinv.reshape(1, NP), b2.reshape(C, 1))
    return out_t[:, :N].T


# speedup vs baseline: 58.0956x; 1.0176x over previous
import functools

import jax
import jax.numpy as jnp
from jax import lax
from jax.experimental import pallas as pl
from jax.experimental.pallas import tpu as pltpu
from jax.experimental.pallas import tpu_sc as plsc

N = 10000
E = 320000
D = 128
H = 32
C = 2

NC = 2
NS = 16
NW = NC * NS
CHUNK = 128
E_PAD = 327680
CPT = E_PAD // (NW * CHUNK)
EPT = CPT * CHUNK
NP = 10240
RPT = NP // NS

_mesh = plsc.VectorSubcoreMesh(core_axis_name="c", subcore_axis_name="s")


@functools.partial(
    pl.kernel,
    mesh=_mesh,
    out_type=jax.ShapeDtypeStruct((NW, NP), jnp.float32),
    scratch_types=[
        pltpu.VMEM((EPT,), jnp.int32),
        pltpu.VMEM((NP,), jnp.float32),
    ],
    compiler_params=pltpu.CompilerParams(needs_layout_passes=False),
)
def _deg_kernel(dst_hbm, out_hbm, dst_v, deg_v):
    c = lax.axis_index("c")
    s = lax.axis_index("s")
    wid = s * NC + c

    pltpu.sync_copy(dst_hbm.at[pl.ds(wid * EPT, EPT)], dst_v)

    zeros = jnp.zeros((16,), jnp.float32)

    def _zero(i, _):
        for u in range(4):
            deg_v[pl.ds(i * 64 + u * 16, 16)] = zeros
        return 0

    lax.fori_loop(0, NP // 64, _zero, 0)

    ones = jnp.full((16,), 1.0, jnp.float32)

    def _count(i, _):
        for u in range(4):
            idx = dst_v[pl.ds(i * 64 + u * 16, 16)]
            plsc.addupdate_scatter(deg_v, [idx], ones)
        return 0

    lax.fori_loop(0, EPT // 64, _count, 0)

    pltpu.sync_copy(deg_v, out_hbm.at[wid])


@functools.partial(
    pl.kernel,
    mesh=_mesh,
    out_type=jax.ShapeDtypeStruct((NC, NP, H), jnp.float32),
    scratch_types=[
        pltpu.VMEM((CPT, CHUNK), jnp.int32),
        pltpu.VMEM((CPT, CHUNK), jnp.int32),
        [pltpu.VMEM((CHUNK, H), jnp.float32) for _ in range(8)],
        pltpu.VMEM_SHARED((NP, H), jnp.float32),
        pltpu.VMEM_SHARED((NP, H), jnp.float32),
        [pltpu.SemaphoreType.DMA for _ in range(8)],
        [pltpu.SemaphoreType.DMA for _ in range(8)],
        pltpu.SemaphoreType.DMA,
    ],
    compiler_params=pltpu.CompilerParams(use_tc_tiling_on_sc=False),
)
def _seg_kernel(table_hbm, src_hbm, dst_hbm, zrows_hbm, out_hbm,
                src_v, dst_v, rows, acc_sh, tab_sh, gsem, ssem, stsem):
    c = lax.axis_index("c")
    s = lax.axis_index("s")
    wid = s * NC + c
    NB = 4

    cp1 = pltpu.async_copy(src_hbm.at[pl.ds(wid * CPT, CPT)], src_v, stsem)
    cp2 = pltpu.async_copy(dst_hbm.at[pl.ds(wid * CPT, CPT)], dst_v, stsem)
    cp3 = pltpu.async_copy(table_hbm.at[pl.ds(s * RPT, RPT)],
                           tab_sh.at[pl.ds(s * RPT, RPT)], stsem)
    cp4 = pltpu.async_copy(zrows_hbm, acc_sh.at[pl.ds(s * RPT, RPT)], stsem)
    cp1.wait()
    cp2.wait()
    cp3.wait()
    cp4.wait()
    plsc.subcore_barrier()

    NR = CPT // (2 * NB)
    bank_a = rows[:NB]
    bank_b = rows[NB:]

    def _gather(buf, gs, j):
        pltpu.async_copy(tab_sh.at[src_v.at[j]], buf, gs)

    def _wait_gather(buf, gs, j):
        pltpu.make_async_copy(tab_sh.at[src_v.at[j]], buf, gs).wait()

    def _scatter(buf, ss, j):
        pltpu.async_copy(buf, acc_sh.at[dst_v.at[j]], ss, add=True)

    def _wait_scatter(buf, ss, j):
        pltpu.make_async_copy(buf, acc_sh.at[dst_v.at[j]], ss).wait()

    for b in range(NB):
        _gather(bank_a[b], gsem[b], b)

    def _round(P, _):
        j0 = 2 * NB * P
        for b in range(NB):
            _wait_gather(bank_a[b], gsem[b], j0 + b)
            _scatter(bank_a[b], ssem[b], j0 + b)
        for b in range(NB):
            @pl.when(P > 0)
            def _():
                _wait_scatter(bank_b[b], ssem[NB + b], j0 - NB + b)
            _gather(bank_b[b], gsem[NB + b], j0 + NB + b)
        for b in range(NB):
            _wait_gather(bank_b[b], gsem[NB + b], j0 + NB + b)
            _scatter(bank_b[b], ssem[NB + b], j0 + NB + b)
        for b in range(NB):
            @pl.when(P < NR - 1)
            def _():
                _wait_scatter(bank_a[b], ssem[b], j0 + b)
                _gather(bank_a[b], gsem[b], j0 + 2 * NB + b)
        return 0

    lax.fori_loop(0, NR, _round, 0)

    for b in range(NB):
        _wait_scatter(bank_a[b], ssem[b], CPT - 2 * NB + b)
        _wait_scatter(bank_b[b], ssem[NB + b], CPT - NB + b)

    plsc.subcore_barrier()
    pltpu.sync_copy(acc_sh.at[pl.ds(s * RPT, RPT)],
                    out_hbm.at[c, pl.ds(s * RPT, RPT)])


@functools.partial(
    pl.kernel,
    mesh=_mesh,
    out_type=jax.ShapeDtypeStruct((NW, C, NP), jnp.float32),
    scratch_types=[
        pltpu.VMEM((EPT,), jnp.int32),
        pltpu.VMEM((EPT,), jnp.int32),
        pltpu.VMEM((NP,), jnp.float32),
        pltpu.VMEM((NP,), jnp.float32),
        pltpu.VMEM((NP,), jnp.float32),
        pltpu.VMEM((NP,), jnp.float32),
        pltpu.SemaphoreType.DMA,
    ],
    compiler_params=pltpu.CompilerParams(needs_layout_passes=False),
)
def _l2_kernel(t2t_hbm, src_hbm, dst_hbm, out_hbm,
               src_v, dst_v, tab0, tab1, acc0, acc1, stsem):
    c = lax.axis_index("c")
    s = lax.axis_index("s")
    wid = s * NC + c

    cp1 = pltpu.async_copy(src_hbm.at[pl.ds(wid * EPT, EPT)], src_v, stsem)
    cp2 = pltpu.async_copy(dst_hbm.at[pl.ds(wid * EPT, EPT)], dst_v, stsem)
    cp3 = pltpu.async_copy(t2t_hbm.at[0], tab0, stsem)
    cp4 = pltpu.async_copy(t2t_hbm.at[1], tab1, stsem)
    cp1.wait()
    cp2.wait()
    cp3.wait()
    cp4.wait()

    zeros = jnp.zeros((16,), jnp.float32)

    def _zero(i, _):
        for u in range(4):
            acc0[pl.ds(i * 64 + u * 16, 16)] = zeros
            acc1[pl.ds(i * 64 + u * 16, 16)] = zeros
        return 0

    lax.fori_loop(0, NP // 64, _zero, 0)

    def _edges(e, _):
        for u in range(4):
            sv = src_v[pl.ds(e * 64 + u * 16, 16)]
            dv = dst_v[pl.ds(e * 64 + u * 16, 16)]
            v0 = plsc.load_gather(tab0, [sv])
            plsc.addupdate_scatter(acc0, [dv], v0)
            v1 = plsc.load_gather(tab1, [sv])
            plsc.addupdate_scatter(acc1, [dv], v1)
        return 0

    lax.fori_loop(0, EPT // 64, _edges, 0)

    pltpu.sync_copy(acc0, out_hbm.at[wid, 0])
    pltpu.sync_copy(acc1, out_hbm.at[wid, 1])


BN = 1024


def _tc_b_body(degp_ref, x_ref, w1_ref, h1s_ref, dinv_ref):
    deg = jnp.sum(degp_ref[...], axis=0) + 1.0
    dinv = lax.rsqrt(deg)
    h = jnp.dot(x_ref[...], w1_ref[...], preferred_element_type=jnp.float32)
    h1s_ref[...] = h * dinv[:, None]
    dinv_ref[...] = dinv[:, None]


def _tc_d_body(acc_ref, h1s_ref, dinv_ref, b1_ref, w2_ref, t2t_ref, *,
               nblock):
    i = pl.program_id(0)
    ssum = acc_ref[0] + acc_ref[1] + h1s_ref[...]
    dinv = dinv_ref[...]
    a = jnp.maximum(dinv * ssum + b1_ref[...], 0.0) * dinv
    row = i * nblock + lax.broadcasted_iota(jnp.int32, (nblock, 1), 0)
    a1s = jnp.where(row < N, a, 0.0)
    t2t_ref[...] = jnp.dot(a1s, w2_ref[...],
                           preferred_element_type=jnp.float32).T


def _tc_e_body(acc_ref, t2t_ref, dinvt_ref, b2_ref, out_ref):
    ssum = jnp.sum(acc_ref[...], axis=0)
    o = dinvt_ref[...] * (ssum + t2t_ref[...]) + b2_ref[...]
    m = jnp.max(o, axis=0, keepdims=True)
    lse = m + jnp.log(jnp.sum(jnp.exp(o - m), axis=0, keepdims=True))
    out_ref[...] = o - lse


def _tc_b(deg_part, x_p, w1):
    grid = NP // BN
    return pl.pallas_call(
        _tc_b_body,
        grid=(grid,),
        in_specs=[
            pl.BlockSpec((NW, BN), lambda i: (0, i)),
            pl.BlockSpec((BN, D), lambda i: (i, 0)),
            pl.BlockSpec((D, H), lambda i: (0, 0)),
        ],
        out_specs=[
            pl.BlockSpec((BN, H), lambda i: (i, 0)),
            pl.BlockSpec((BN, 1), lambda i: (i, 0)),
        ],
        out_shape=[
            jax.ShapeDtypeStruct((NP, H), jnp.float32),
            jax.ShapeDtypeStruct((NP, 1), jnp.float32),
        ],
    )(deg_part, x_p, w1)


def _tc_d(acc1, h1s, dinv, b1, w2):
    grid = NP // BN
    return pl.pallas_call(
        functools.partial(_tc_d_body, nblock=BN),
        grid=(grid,),
        in_specs=[
            pl.BlockSpec((NC, BN, H), lambda i: (0, i, 0)),
            pl.BlockSpec((BN, H), lambda i: (i, 0)),
            pl.BlockSpec((BN, 1), lambda i: (i, 0)),
            pl.BlockSpec((1, H), lambda i: (0, 0)),
            pl.BlockSpec((H, C), lambda i: (0, 0)),
        ],
        out_specs=pl.BlockSpec((C, BN), lambda i: (0, i)),
        out_shape=jax.ShapeDtypeStruct((C, NP), jnp.float32),
    )(acc1, h1s, dinv, b1, w2)


def _tc_e(acc2p, t2t, dinvt, b2):
    grid = NP // BN
    return pl.pallas_call(
        _tc_e_body,
        grid=(grid,),
        in_specs=[
            pl.BlockSpec((NW, C, BN), lambda i: (0, 0, i)),
            pl.BlockSpec((C, BN), lambda i: (0, i)),
            pl.BlockSpec((1, BN), lambda i: (0, i)),
            pl.BlockSpec((C, 1), lambda i: (0, 0)),
        ],
        out_specs=pl.BlockSpec((C, BN), lambda i: (0, i)),
        out_shape=jax.ShapeDtypeStruct((C, NP), jnp.float32),
    )(acc2p, t2t, dinvt, b2)


@jax.jit
def kernel(x, edge_index, W1, b1, W2, b2):
    src = edge_index[0]
    dst = edge_index[1]
    pad = jnp.full((E_PAD - E,), N, jnp.int32)
    src_p = jnp.concatenate([src, pad]).reshape(E_PAD // CHUNK, CHUNK)
    dst_p = jnp.concatenate([dst, pad]).reshape(E_PAD // CHUNK, CHUNK)
    src_flat = src_p.reshape(E_PAD)
    dst_flat = dst_p.reshape(E_PAD)
    x_p = jnp.pad(x, ((0, NP - N), (0, 0)))
    zrows = jnp.zeros((RPT, H), jnp.float32)

    deg_part = _deg_kernel(dst_flat)
    h1s, dinv = _tc_b(deg_part, x_p, W1)
    acc1 = _seg_kernel(h1s, src_p, dst_p, zrows)
    t2t = _tc_d(acc1, h1s, dinv, b1.reshape(1, H), W2)
    acc2p = _l2_kernel(t2t, src_flat, dst_flat)
    out_t = _tc_e(acc2p, t2t, dinv.reshape(1, NP), b2.reshape(C, 1))
    return out_t[:, :N].T
